# baseline (speedup 1.0000x reference)
import jax
import jax.numpy as jnp
from jax import lax
from jax.experimental import pallas as pl
from jax.experimental.pallas import tpu as pltpu
from jax.experimental.pallas import tpu_sc as plsc

N = 10000
NP = 10240
E = 320000
K = 11
NC, NS = 2, 16
EPW = E // NS
C = 80
NCHUNK = EPW // C
RPT = NP // NS
RING = 4
NG = NCHUNK // RING
LEFT = NCHUNK - NG * RING


def _make_segsum(Dh, with_deg):
    mesh = plsc.VectorSubcoreMesh(core_axis_name="c", subcore_axis_name="s")
    out_type = [jax.ShapeDtypeStruct((NP, NC, Dh), jnp.float32)]
    scratch = [
        pltpu.VMEM((NCHUNK, C), jnp.int32),
        pltpu.VMEM((NCHUNK, C), jnp.int32),
        pltpu.VMEM_SHARED((NP, Dh), jnp.float32),
    ]
    scratch += [pltpu.VMEM((C, Dh), jnp.float32) for _ in range(RING)]
    scratch += [pltpu.SemaphoreType.DMA for _ in range(2 * RING)]
    if with_deg:
        out_type.append(jax.ShapeDtypeStruct((NP,), jnp.float32))
        scratch += [
            pltpu.VMEM((C,), jnp.float32),
            pltpu.VMEM((RPT,), jnp.float32),
            pltpu.VMEM_SHARED((NP,), jnp.float32),
            pltpu.SemaphoreType.DMA,
        ]

    def body(xf_hbm, src_hbm, srcb_hbm, dst_hbm, *rest):
        h_out = rest[0]
        rest = rest[1:]
        if with_deg:
            deg_out = rest[0]
            rest = rest[1:]
        src_all, dst_all, h_sh = rest[0], rest[1], rest[2]
        bufs = rest[3:3 + RING]
        gsem = rest[3 + RING:3 + 2 * RING]
        ssem = rest[3 + 2 * RING:3 + 3 * RING]
        if with_deg:
            ones_v, dz_v, deg_sh, dsem = rest[3 + 3 * RING:]
        c = lax.axis_index("c")
        s = lax.axis_index("s")
        row0 = s * RPT

        zero16 = jnp.zeros((16,), jnp.float32)
        b0 = bufs[0]

        @pl.loop(0, C)
        def _zero_b0(r):
            for j in range(Dh // 16):
                b0[r, pl.ds(j * 16, 16)] = zero16

        @pl.when(c == 0)
        def _load_src0():
            pltpu.sync_copy(src_hbm.at[s], src_all)

        @pl.when(c == 1)
        def _load_src1():
            pltpu.sync_copy(srcb_hbm.at[s], src_all)

        pltpu.sync_copy(dst_hbm.at[s], dst_all)

        @pl.loop(0, RPT // C)
        def _zero_h(k):
            pltpu.sync_copy(b0, h_sh.at[pl.ds(row0 + k * C, C)])

        if with_deg:
            @pl.loop(0, RPT // 16)
            def _zero_dz(i):
                dz_v[pl.ds(i * 16, 16)] = zero16

            for j in range(C // 16):
                ones_v[pl.ds(j * 16, 16)] = jnp.ones((16,), jnp.float32)
            pltpu.sync_copy(dz_v, deg_sh.at[pl.ds(row0, RPT)])

        plsc.subcore_barrier()

        def g_issue(i, buf, sem):
            pltpu.async_copy(xf_hbm.at[src_all.at[i]], buf, sem)

        def g_wait(i, buf, sem):
            pltpu.make_async_copy(xf_hbm.at[src_all.at[i]], buf, sem).wait()

        def s_issue(i, buf, sem):
            pltpu.async_copy(buf, h_sh.at[dst_all.at[i]], sem, add=True)

        def s_wait(i, buf, sem):
            pltpu.make_async_copy(buf, h_sh.at[dst_all.at[i]], sem).wait()

        def d_issue(i):
            pltpu.async_copy(ones_v, deg_sh.at[dst_all.at[i]], dsem, add=True)

        def d_wait(i):
            pltpu.make_async_copy(ones_v, deg_sh.at[dst_all.at[i]], dsem).wait()

        for l in range(LEFT):
            e = NCHUNK - 1 - l
            g_issue(e, bufs[l], gsem[l])
            g_wait(e, bufs[l], gsem[l])
            s_issue(e, bufs[l], ssem[l])
            if with_deg:
                @pl.when(c == 0)
                def _dl():
                    d_issue(e)
                    d_wait(e)
            s_wait(e, bufs[l], ssem[l])

        for j in range(RING):
            g_issue(j, bufs[j], gsem[j])

        @pl.loop(0, NG)
        def _groups(g):
            e0 = g * RING
            for j in range(RING):
                e = e0 + j
                g_wait(e, bufs[j], gsem[j])
                s_issue(e, bufs[j], ssem[j])
                if with_deg:
                    @pl.when(c == 0)
                    def _di():
                        d_issue(e)
            for j in range(RING):
                e = e0 + j
                s_wait(e, bufs[j], ssem[j])

                @pl.when(g < NG - 1)
                def _next_gather():
                    g_issue(e + RING, bufs[j], gsem[j])
            if with_deg:
                @pl.when(c == 0)
                def _dd():
                    for j in range(RING):
                        d_wait(e0 + j)

        plsc.subcore_barrier()

        pltpu.sync_copy(h_sh.at[pl.ds(row0, RPT)],
                        h_out.at[pl.ds(row0, RPT), c])
        if with_deg:
            @pl.when(c == 0)
            def _deg_wb():
                pltpu.sync_copy(deg_sh.at[pl.ds(row0, RPT)],
                                deg_out.at[pl.ds(row0, RPT)])

    return pl.kernel(body, out_type=tuple(out_type), mesh=mesh,
                     scratch_types=tuple(scratch),
                     compiler_params=pltpu.CompilerParams(
                         use_tc_tiling_on_sc=False))


_segsum_cache = {}


def _segsum(Dh, with_deg):
    key = (Dh, with_deg)
    if key not in _segsum_cache:
        _segsum_cache[key] = _make_segsum(Dh, with_deg)
    return _segsum_cache[key]


def _mfconv_tc(h, xin, wlf, blf, wrf, degcol, hout, relu):
    B = 1024
    din = xin.shape[1]
    kh = wlf.shape[1]
    grid = (pl.cdiv(N, B),)

    def tc_body(h0, xr, wl, bl, wr, d0, out):
        r = jnp.dot(h0[...], wl[...], preferred_element_type=jnp.float32)
        r += jnp.dot(xr[...], wr[...], preferred_element_type=jnp.float32)
        r += bl[...]
        deg = jnp.minimum(d0[...], float(K - 1))
        grp = (lax.broadcasted_iota(jnp.int32, (1, kh), 1) // hout
               ).astype(jnp.float32)
        m = r * (deg == grp).astype(jnp.float32)
        if hout % 128 == 0:
            acc = m[:, 0:hout]
            for d in range(1, K):
                acc += m[:, d * hout:(d + 1) * hout]
        else:
            row = lax.broadcasted_iota(jnp.int32, (kh, hout), 0)
            col = lax.broadcasted_iota(jnp.int32, (kh, hout), 1)
            g = (row % hout == col).astype(jnp.float32)
            acc = jnp.dot(m, g, preferred_element_type=jnp.float32)
        out[...] = jnp.maximum(acc, 0.0) if relu else acc

    return pl.pallas_call(
        tc_body,
        grid=grid,
        in_specs=[
            pl.BlockSpec((B, h.shape[1]), lambda i: (i, 0)),
            pl.BlockSpec((B, din), lambda i: (i, 0)),
            pl.BlockSpec((din, kh), lambda i: (0, 0)),
            pl.BlockSpec((1, kh), lambda i: (0, 0)),
            pl.BlockSpec((din, kh), lambda i: (0, 0)),
            pl.BlockSpec((B, 1), lambda i: (i, 0)),
        ],
        out_specs=pl.BlockSpec((B, hout), lambda i: (i, 0)),
        out_shape=jax.ShapeDtypeStruct((N, hout), jnp.float32),
    )(h, xin, wlf, blf, wrf, degcol)


def kernel(x, edge_index, Wl1, bl1, Wr1, Wl2, bl2, Wr2):
    ei = edge_index.astype(jnp.int32)
    src3 = ei[0].reshape(NS, NCHUNK, C)
    src3b = src3 + N
    dst3 = ei[1].reshape(NS, NCHUNK, C)

    wl1f = Wl1.transpose(1, 0, 2).reshape(128, K * 32)
    wr1f = Wr1.transpose(1, 0, 2).reshape(128, K * 32)
    bl1f = bl1.reshape(1, K * 32)
    wl2f = Wl2.transpose(1, 0, 2).reshape(32, K * 128)
    wr2f = Wr2.transpose(1, 0, 2).reshape(32, K * 128)
    bl2f = bl2.reshape(1, K * 128)

    xf = jnp.concatenate([x[:, :64], x[:, 64:]], axis=0)
    h1, deg = _segsum(64, True)(xf, src3, src3b, dst3)
    degcol = deg.reshape(NP, 1)
    o1 = _mfconv_tc(h1.reshape(NP, 128), x, wl1f, bl1f, wr1f, degcol,
                    32, relu=True)
    o1f = jnp.concatenate([o1[:, :16], o1[:, 16:]], axis=0)
    (h2,) = _segsum(16, False)(o1f, src3, src3b, dst3)
    out = _mfconv_tc(h2.reshape(NP, 32), o1, wl2f, bl2f, wr2f, degcol,
                     128, relu=False)
    return out


# speedup vs baseline: 1.3767x; 1.3767x over previous
import jax
import jax.numpy as jnp
from jax import lax
from jax.experimental import pallas as pl
from jax.experimental.pallas import tpu as pltpu
from jax.experimental.pallas import tpu_sc as plsc

N = 10000
NP = 10240
E = 320000
K = 11
NC, NS = 2, 16
NW = NC * NS
EPW = E // NW
C = 80
NCHUNK = EPW // C
RPT = NP // NS


def _make_segsum(D, with_deg, ring):
    ng = NCHUNK // ring
    left = NCHUNK - ng * ring
    mesh = plsc.VectorSubcoreMesh(core_axis_name="c", subcore_axis_name="s")
    out_type = [jax.ShapeDtypeStruct((NC, NP, D), jnp.float32)]
    scratch = [
        pltpu.VMEM((NCHUNK, C), jnp.int32),
        pltpu.VMEM((NCHUNK, C), jnp.int32),
        pltpu.VMEM_SHARED((NP, D), jnp.float32),
    ]
    scratch += [pltpu.VMEM((C, D), jnp.float32) for _ in range(ring)]
    scratch += [pltpu.SemaphoreType.DMA for _ in range(2 * ring)]
    if with_deg:
        out_type.append(jax.ShapeDtypeStruct((NC, NP), jnp.float32))
        scratch += [
            pltpu.VMEM((C,), jnp.float32),
            pltpu.VMEM((RPT,), jnp.float32),
            pltpu.VMEM_SHARED((NP,), jnp.float32),
            pltpu.SemaphoreType.DMA,
        ]

    def body(x_hbm, src_hbm, dst_hbm, *rest):
        h_out = rest[0]
        rest = rest[1:]
        if with_deg:
            deg_out = rest[0]
            rest = rest[1:]
        src_all, dst_all, h_sh = rest[0], rest[1], rest[2]
        bufs = rest[3:3 + ring]
        gsem = rest[3 + ring:3 + 2 * ring]
        ssem = rest[3 + 2 * ring:3 + 3 * ring]
        if with_deg:
            ones_v, dz_v, deg_sh, dsem = rest[3 + 3 * ring:]
        c = lax.axis_index("c")
        s = lax.axis_index("s")
        wid = c * NS + s
        row0 = s * RPT

        zero16 = jnp.zeros((16,), jnp.float32)
        b0 = bufs[0]

        @pl.loop(0, C)
        def _zero_b0(r):
            for j in range(D // 16):
                b0[r, pl.ds(j * 16, 16)] = zero16

        pltpu.sync_copy(src_hbm.at[wid], src_all)
        pltpu.sync_copy(dst_hbm.at[wid], dst_all)

        @pl.loop(0, RPT // C)
        def _zero_h(k):
            pltpu.sync_copy(b0, h_sh.at[pl.ds(row0 + k * C, C)])

        if with_deg:
            @pl.loop(0, RPT // 16)
            def _zero_dz(i):
                dz_v[pl.ds(i * 16, 16)] = zero16

            for j in range(C // 16):
                ones_v[pl.ds(j * 16, 16)] = jnp.ones((16,), jnp.float32)
            pltpu.sync_copy(dz_v, deg_sh.at[pl.ds(row0, RPT)])

        plsc.subcore_barrier()

        def g_issue(i, buf, sem):
            pltpu.async_copy(x_hbm.at[src_all.at[i]], buf, sem)

        def g_wait(i, buf, sem):
            pltpu.make_async_copy(x_hbm.at[src_all.at[i]], buf, sem).wait()

        def s_issue(i, buf, sem):
            pltpu.async_copy(buf, h_sh.at[dst_all.at[i]], sem, add=True)

        def s_wait(i, buf, sem):
            pltpu.make_async_copy(buf, h_sh.at[dst_all.at[i]], sem).wait()

        def d_issue(i):
            pltpu.async_copy(ones_v, deg_sh.at[dst_all.at[i]], dsem, add=True)

        def d_wait(i):
            pltpu.make_async_copy(ones_v, deg_sh.at[dst_all.at[i]], dsem).wait()

        for l in range(left):
            e = NCHUNK - 1 - l
            g_issue(e, bufs[l], gsem[l])
            g_wait(e, bufs[l], gsem[l])
            s_issue(e, bufs[l], ssem[l])
            if with_deg:
                d_issue(e)
                d_wait(e)
            s_wait(e, bufs[l], ssem[l])

        for j in range(ring):
            g_issue(j, bufs[j], gsem[j])

        @pl.loop(0, ng)
        def _groups(g):
            e0 = g * ring
            for j in range(ring):
                e = e0 + j
                g_wait(e, bufs[j], gsem[j])
                s_issue(e, bufs[j], ssem[j])
                if with_deg:
                    d_issue(e)
            for j in range(ring):
                e = e0 + j
                s_wait(e, bufs[j], ssem[j])

                @pl.when(g < ng - 1)
                def _next_gather():
                    g_issue(e + ring, bufs[j], gsem[j])
            if with_deg:
                for j in range(ring):
                    d_wait(e0 + j)

        plsc.subcore_barrier()

        pltpu.sync_copy(h_sh.at[pl.ds(row0, RPT)], h_out.at[c, pl.ds(row0, RPT)])
        if with_deg:
            pltpu.sync_copy(deg_sh.at[pl.ds(row0, RPT)],
                            deg_out.at[c, pl.ds(row0, RPT)])

    return pl.kernel(body, out_type=tuple(out_type), mesh=mesh,
                     scratch_types=tuple(scratch),
                     compiler_params=pltpu.CompilerParams(
                         use_tc_tiling_on_sc=False))


_segsum_cache = {}


def _segsum(D, with_deg, ring):
    key = (D, with_deg, ring)
    if key not in _segsum_cache:
        _segsum_cache[key] = _make_segsum(D, with_deg, ring)
    return _segsum_cache[key]


def _mfconv_tc(hp, xin, wlf, blf, wrf, degcol, hout, relu):
    B = 1024
    din = xin.shape[1]
    kh = wlf.shape[1]
    grid = (pl.cdiv(N, B),)

    def tc_body(h0, h1, xr, wl, bl, wr, d0, d1, out):
        h = h0[0] + h1[0]
        r = jnp.dot(h, wl[...], preferred_element_type=jnp.float32)
        r += jnp.dot(xr[...], wr[...], preferred_element_type=jnp.float32)
        r += bl[...]
        deg = jnp.minimum(d0[0] + d1[0], float(K - 1))
        grp = (lax.broadcasted_iota(jnp.int32, (1, kh), 1) // hout
               ).astype(jnp.float32)
        m = r * (deg == grp).astype(jnp.float32)
        if hout % 128 == 0:
            acc = m[:, 0:hout]
            for d in range(1, K):
                acc += m[:, d * hout:(d + 1) * hout]
        else:
            row = lax.broadcasted_iota(jnp.int32, (kh, hout), 0)
            col = lax.broadcasted_iota(jnp.int32, (kh, hout), 1)
            g = (row % hout == col).astype(jnp.float32)
            acc = jnp.dot(m, g, preferred_element_type=jnp.float32)
        out[...] = jnp.maximum(acc, 0.0) if relu else acc

    return pl.pallas_call(
        tc_body,
        grid=grid,
        in_specs=[
            pl.BlockSpec((1, B, hp.shape[2]), lambda i: (0, i, 0)),
            pl.BlockSpec((1, B, hp.shape[2]), lambda i: (1, i, 0)),
            pl.BlockSpec((B, din), lambda i: (i, 0)),
            pl.BlockSpec((din, kh), lambda i: (0, 0)),
            pl.BlockSpec((1, kh), lambda i: (0, 0)),
            pl.BlockSpec((din, kh), lambda i: (0, 0)),
            pl.BlockSpec((1, B, 1), lambda i: (0, i, 0)),
            pl.BlockSpec((1, B, 1), lambda i: (1, i, 0)),
        ],
        out_specs=pl.BlockSpec((B, hout), lambda i: (i, 0)),
        out_shape=jax.ShapeDtypeStruct((N, hout), jnp.float32),
    )(hp, hp, xin, wlf, blf, wrf, degcol, degcol)


def kernel(x, edge_index, Wl1, bl1, Wr1, Wl2, bl2, Wr2):
    ei = edge_index.astype(jnp.int32)
    src = ei[0].reshape(NW, NCHUNK, C)
    dst = ei[1].reshape(NW, NCHUNK, C)

    wl1f = Wl1.transpose(1, 0, 2).reshape(128, K * 32)
    wr1f = Wr1.transpose(1, 0, 2).reshape(128, K * 32)
    bl1f = bl1.reshape(1, K * 32)
    wl2f = Wl2.transpose(1, 0, 2).reshape(32, K * 128)
    wr2f = Wr2.transpose(1, 0, 2).reshape(32, K * 128)
    bl2f = bl2.reshape(1, K * 128)

    hp1, degp = _segsum(128, True, 2)(x, src, dst)
    degcol = degp.reshape(NC, NP, 1)
    o1 = _mfconv_tc(hp1, x, wl1f, bl1f, wr1f, degcol, 32, relu=True)
    (hp2,) = _segsum(32, False, 4)(o1, src, dst)
    out = _mfconv_tc(hp2, o1, wl2f, bl2f, wr2f, degcol, 128, relu=False)
    return out


# speedup vs baseline: 1.4823x; 1.0767x over previous
import jax
import jax.numpy as jnp
from jax import lax
from jax.experimental import pallas as pl
from jax.experimental.pallas import tpu as pltpu
from jax.experimental.pallas import tpu_sc as plsc

N = 10000
NP = 10240
E = 320000
K = 11
NC, NS = 2, 16
NW = NC * NS
EPW = E // NW
C = 80
NCHUNK = EPW // C
RPT = NP // NS


def _make_segsum(D, with_deg, ring):
    ng = NCHUNK // ring
    left = NCHUNK - ng * ring
    mesh = plsc.VectorSubcoreMesh(core_axis_name="c", subcore_axis_name="s")
    out_type = [jax.ShapeDtypeStruct((NC, NP, D), jnp.float32)]
    scratch = [
        pltpu.VMEM((NCHUNK, C), jnp.int32),
        pltpu.VMEM((NCHUNK, C), jnp.int32),
        pltpu.VMEM_SHARED((NP, D), jnp.float32),
    ]
    scratch += [pltpu.VMEM((C, D), jnp.float32) for _ in range(ring)]
    scratch += [pltpu.SemaphoreType.DMA for _ in range(2 * ring)]
    if with_deg:
        out_type.append(jax.ShapeDtypeStruct((NC, NP), jnp.float32))
        scratch += [
            pltpu.VMEM((C,), jnp.float32),
            pltpu.VMEM((RPT,), jnp.float32),
            pltpu.VMEM_SHARED((NP,), jnp.float32),
            pltpu.SemaphoreType.DMA,
        ]

    def body(x_hbm, src_hbm, dst_hbm, *rest):
        h_out = rest[0]
        rest = rest[1:]
        if with_deg:
            deg_out = rest[0]
            rest = rest[1:]
        src_all, dst_all, h_sh = rest[0], rest[1], rest[2]
        bufs = rest[3:3 + ring]
        gsem = rest[3 + ring:3 + 2 * ring]
        ssem = rest[3 + 2 * ring:3 + 3 * ring]
        if with_deg:
            ones_v, dz_v, deg_sh, dsem = rest[3 + 3 * ring:]
        c = lax.axis_index("c")
        s = lax.axis_index("s")
        wid = c * NS + s
        row0 = s * RPT

        zero16 = jnp.zeros((16,), jnp.float32)
        b0 = bufs[0]

        @pl.loop(0, C)
        def _zero_b0(r):
            for j in range(D // 16):
                b0[r, pl.ds(j * 16, 16)] = zero16

        pltpu.sync_copy(src_hbm.at[wid], src_all)
        pltpu.sync_copy(dst_hbm.at[wid], dst_all)

        @pl.loop(0, RPT // C)
        def _zero_h(k):
            pltpu.sync_copy(b0, h_sh.at[pl.ds(row0 + k * C, C)])

        if with_deg:
            @pl.loop(0, RPT // 16)
            def _zero_dz(i):
                dz_v[pl.ds(i * 16, 16)] = zero16

            for j in range(C // 16):
                ones_v[pl.ds(j * 16, 16)] = jnp.ones((16,), jnp.float32)
            pltpu.sync_copy(dz_v, deg_sh.at[pl.ds(row0, RPT)])

        plsc.subcore_barrier()

        def g_issue(i, buf, sem):
            pltpu.async_copy(x_hbm.at[src_all.at[i]], buf, sem)

        def g_wait(i, buf, sem):
            pltpu.make_async_copy(x_hbm.at[src_all.at[i]], buf, sem).wait()

        def s_issue(i, buf, sem):
            pltpu.async_copy(buf, h_sh.at[dst_all.at[i]], sem, add=True)

        def s_wait(i, buf, sem):
            pltpu.make_async_copy(buf, h_sh.at[dst_all.at[i]], sem).wait()

        def d_issue(i):
            pltpu.async_copy(ones_v, deg_sh.at[dst_all.at[i]], dsem, add=True)

        def d_wait(i):
            pltpu.make_async_copy(ones_v, deg_sh.at[dst_all.at[i]], dsem).wait()

        if ring == 2:
            npair = (NCHUNK - 1) // 2
            buf0, buf1 = bufs
            gs0, gs1 = gsem
            ss0, ss1 = ssem
            g_issue(0, buf0, gs0)
            g_issue(1, buf1, gs1)

            @pl.loop(0, npair)
            def _pairs(k):
                e = k * 2
                g_wait(e, buf0, gs0)
                s_issue(e, buf0, ss0)
                if with_deg:
                    d_issue(e)
                s_wait(e, buf0, ss0)
                g_issue(e + 2, buf0, gs0)
                g_wait(e + 1, buf1, gs1)
                s_issue(e + 1, buf1, ss1)
                if with_deg:
                    d_issue(e + 1)
                s_wait(e + 1, buf1, ss1)

                @pl.when(k < npair - 1)
                def _prefetch_odd():
                    g_issue(e + 3, buf1, gs1)

                if with_deg:
                    d_wait(e)
                    d_wait(e + 1)

            last = NCHUNK - 1
            g_wait(last, buf0, gs0)
            s_issue(last, buf0, ss0)
            if with_deg:
                d_issue(last)
            s_wait(last, buf0, ss0)
            if with_deg:
                d_wait(last)
        else:
            for l in range(left):
                e = NCHUNK - 1 - l
                g_issue(e, bufs[l], gsem[l])
                g_wait(e, bufs[l], gsem[l])
                s_issue(e, bufs[l], ssem[l])
                if with_deg:
                    d_issue(e)
                    d_wait(e)
                s_wait(e, bufs[l], ssem[l])

            for j in range(ring):
                g_issue(j, bufs[j], gsem[j])

            @pl.loop(0, ng)
            def _groups(g):
                e0 = g * ring
                for j in range(ring):
                    e = e0 + j
                    g_wait(e, bufs[j], gsem[j])
                    s_issue(e, bufs[j], ssem[j])
                    if with_deg:
                        d_issue(e)
                for j in range(ring):
                    e = e0 + j
                    s_wait(e, bufs[j], ssem[j])

                    @pl.when(g < ng - 1)
                    def _next_gather():
                        g_issue(e + ring, bufs[j], gsem[j])
                if with_deg:
                    for j in range(ring):
                        d_wait(e0 + j)

        plsc.subcore_barrier()

        pltpu.sync_copy(h_sh.at[pl.ds(row0, RPT)], h_out.at[c, pl.ds(row0, RPT)])
        if with_deg:
            pltpu.sync_copy(deg_sh.at[pl.ds(row0, RPT)],
                            deg_out.at[c, pl.ds(row0, RPT)])

    return pl.kernel(body, out_type=tuple(out_type), mesh=mesh,
                     scratch_types=tuple(scratch),
                     compiler_params=pltpu.CompilerParams(
                         use_tc_tiling_on_sc=False))


_segsum_cache = {}


def _segsum(D, with_deg, ring):
    key = (D, with_deg, ring)
    if key not in _segsum_cache:
        _segsum_cache[key] = _make_segsum(D, with_deg, ring)
    return _segsum_cache[key]


def _mfconv_tc(hp, xin, wlf, blf, wrf, degcol, hout, relu):
    B = 1024
    din = xin.shape[1]
    kh = wlf.shape[1]
    grid = (pl.cdiv(N, B),)

    def tc_body(h0, h1, xr, wl, bl, wr, d0, d1, out):
        h = h0[0] + h1[0]
        r = jnp.dot(h, wl[...], preferred_element_type=jnp.float32)
        r += jnp.dot(xr[...], wr[...], preferred_element_type=jnp.float32)
        r += bl[...]
        deg = jnp.minimum(d0[0] + d1[0], float(K - 1))
        grp = (lax.broadcasted_iota(jnp.int32, (1, kh), 1) // hout
               ).astype(jnp.float32)
        m = r * (deg == grp).astype(jnp.float32)
        if hout % 128 == 0:
            acc = m[:, 0:hout]
            for d in range(1, K):
                acc += m[:, d * hout:(d + 1) * hout]
        else:
            row = lax.broadcasted_iota(jnp.int32, (kh, hout), 0)
            col = lax.broadcasted_iota(jnp.int32, (kh, hout), 1)
            g = (row % hout == col).astype(jnp.float32)
            acc = jnp.dot(m, g, preferred_element_type=jnp.float32)
        out[...] = jnp.maximum(acc, 0.0) if relu else acc

    return pl.pallas_call(
        tc_body,
        grid=grid,
        in_specs=[
            pl.BlockSpec((1, B, hp.shape[2]), lambda i: (0, i, 0)),
            pl.BlockSpec((1, B, hp.shape[2]), lambda i: (1, i, 0)),
            pl.BlockSpec((B, din), lambda i: (i, 0)),
            pl.BlockSpec((din, kh), lambda i: (0, 0)),
            pl.BlockSpec((1, kh), lambda i: (0, 0)),
            pl.BlockSpec((din, kh), lambda i: (0, 0)),
            pl.BlockSpec((1, B, 1), lambda i: (0, i, 0)),
            pl.BlockSpec((1, B, 1), lambda i: (1, i, 0)),
        ],
        out_specs=pl.BlockSpec((B, hout), lambda i: (i, 0)),
        out_shape=jax.ShapeDtypeStruct((N, hout), jnp.float32),
    )(hp, hp, xin, wlf, blf, wrf, degcol, degcol)


def kernel(x, edge_index, Wl1, bl1, Wr1, Wl2, bl2, Wr2):
    ei = edge_index.astype(jnp.int32)
    src = ei[0].reshape(NW, NCHUNK, C)
    dst = ei[1].reshape(NW, NCHUNK, C)

    wl1f = Wl1.transpose(1, 0, 2).reshape(128, K * 32)
    wr1f = Wr1.transpose(1, 0, 2).reshape(128, K * 32)
    bl1f = bl1.reshape(1, K * 32)
    wl2f = Wl2.transpose(1, 0, 2).reshape(32, K * 128)
    wr2f = Wr2.transpose(1, 0, 2).reshape(32, K * 128)
    bl2f = bl2.reshape(1, K * 128)

    hp1, degp = _segsum(128, True, 2)(x, src, dst)
    degcol = degp.reshape(NC, NP, 1)
    o1 = _mfconv_tc(hp1, x, wl1f, bl1f, wr1f, degcol, 32, relu=True)
    (hp2,) = _segsum(32, False, 4)(o1, src, dst)
    out = _mfconv_tc(hp2, o1, wl2f, bl2f, wr2f, degcol, 128, relu=False)
    return out


# speedup vs baseline: 1.5187x; 1.0245x over previous
import jax
import jax.numpy as jnp
from jax import lax
from jax.experimental import pallas as pl
from jax.experimental.pallas import tpu as pltpu
from jax.experimental.pallas import tpu_sc as plsc

N = 10000
NP = 10240
E = 320000
K = 11
NC, NS = 2, 16
NW = NC * NS
EPW = E // NW
C = 80
NCHUNK = EPW // C
RPT = NP // NS


def _make_segsum(D, with_deg, ring):
    ng = NCHUNK // ring
    left = NCHUNK - ng * ring
    mesh = plsc.VectorSubcoreMesh(core_axis_name="c", subcore_axis_name="s")
    out_type = [jax.ShapeDtypeStruct((NC, NP, D), jnp.float32)]
    scratch = [
        pltpu.VMEM((NCHUNK, C), jnp.int32),
        pltpu.VMEM((NCHUNK, C), jnp.int32),
        pltpu.VMEM_SHARED((NP, D), jnp.float32),
    ]
    scratch += [pltpu.VMEM((C, D), jnp.float32) for _ in range(ring)]
    scratch += [pltpu.SemaphoreType.DMA for _ in range(2 * ring)]
    if with_deg:
        out_type.append(jax.ShapeDtypeStruct((NC, NP), jnp.float32))
        scratch += [
            pltpu.VMEM((C,), jnp.float32),
            pltpu.VMEM((RPT,), jnp.float32),
            pltpu.VMEM_SHARED((NP,), jnp.float32),
            pltpu.SemaphoreType.DMA,
        ]

    def body(x_hbm, src_hbm, dst_hbm, *rest):
        h_out = rest[0]
        rest = rest[1:]
        if with_deg:
            deg_out = rest[0]
            rest = rest[1:]
        src_all, dst_all, h_sh = rest[0], rest[1], rest[2]
        bufs = rest[3:3 + ring]
        gsem = rest[3 + ring:3 + 2 * ring]
        ssem = rest[3 + 2 * ring:3 + 3 * ring]
        if with_deg:
            ones_v, dz_v, deg_sh, dsem = rest[3 + 3 * ring:]
        c = lax.axis_index("c")
        s = lax.axis_index("s")
        wid = c * NS + s
        row0 = s * RPT

        zero16 = jnp.zeros((16,), jnp.float32)
        b0 = bufs[0]

        @pl.loop(0, C)
        def _zero_b0(r):
            for j in range(D // 16):
                b0[r, pl.ds(j * 16, 16)] = zero16

        pltpu.sync_copy(src_hbm.at[wid], src_all)
        pltpu.sync_copy(dst_hbm.at[wid], dst_all)

        @pl.loop(0, RPT // C)
        def _zero_h(k):
            pltpu.sync_copy(b0, h_sh.at[pl.ds(row0 + k * C, C)])

        if with_deg:
            @pl.loop(0, RPT // 16)
            def _zero_dz(i):
                dz_v[pl.ds(i * 16, 16)] = zero16

            for j in range(C // 16):
                ones_v[pl.ds(j * 16, 16)] = jnp.ones((16,), jnp.float32)
            pltpu.sync_copy(dz_v, deg_sh.at[pl.ds(row0, RPT)])

        plsc.subcore_barrier()

        def g_issue(i, buf, sem):
            pltpu.async_copy(x_hbm.at[src_all.at[i]], buf, sem)

        def g_wait(i, buf, sem):
            pltpu.make_async_copy(x_hbm.at[src_all.at[i]], buf, sem).wait()

        def s_issue(i, buf, sem):
            pltpu.async_copy(buf, h_sh.at[dst_all.at[i]], sem, add=True)

        def s_wait(i, buf, sem):
            pltpu.make_async_copy(buf, h_sh.at[dst_all.at[i]], sem).wait()

        def d_issue(i):
            pltpu.async_copy(ones_v, deg_sh.at[dst_all.at[i]], dsem, add=True)

        def d_wait(i):
            pltpu.make_async_copy(ones_v, deg_sh.at[dst_all.at[i]], dsem).wait()

        if ring == 2:
            npair = (NCHUNK - 1) // 2
            buf0, buf1 = bufs
            gs0, gs1 = gsem
            ss0, ss1 = ssem
            g_issue(0, buf0, gs0)
            g_issue(1, buf1, gs1)

            @pl.loop(0, npair)
            def _pairs(k):
                e = k * 2
                g_wait(e, buf0, gs0)
                s_issue(e, buf0, ss0)
                if with_deg:
                    d_issue(e)
                s_wait(e, buf0, ss0)
                g_issue(e + 2, buf0, gs0)
                g_wait(e + 1, buf1, gs1)
                s_issue(e + 1, buf1, ss1)
                if with_deg:
                    d_issue(e + 1)
                s_wait(e + 1, buf1, ss1)

                @pl.when(k < npair - 1)
                def _prefetch_odd():
                    g_issue(e + 3, buf1, gs1)

                if with_deg:
                    d_wait(e)
                    d_wait(e + 1)

            last = NCHUNK - 1
            g_wait(last, buf0, gs0)
            s_issue(last, buf0, ss0)
            if with_deg:
                d_issue(last)
            s_wait(last, buf0, ss0)
            if with_deg:
                d_wait(last)
        else:
            for l in range(left):
                e = NCHUNK - 1 - l
                g_issue(e, bufs[l], gsem[l])
                g_wait(e, bufs[l], gsem[l])
                s_issue(e, bufs[l], ssem[l])
                if with_deg:
                    d_issue(e)
                    d_wait(e)
                s_wait(e, bufs[l], ssem[l])

            for j in range(ring):
                g_issue(j, bufs[j], gsem[j])

            @pl.loop(0, ng)
            def _groups(g):
                e0 = g * ring
                for j in range(ring):
                    e = e0 + j
                    g_wait(e, bufs[j], gsem[j])
                    s_issue(e, bufs[j], ssem[j])
                    if with_deg:
                        d_issue(e)
                for j in range(ring):
                    e = e0 + j
                    s_wait(e, bufs[j], ssem[j])

                    @pl.when(g < ng - 1)
                    def _next_gather():
                        g_issue(e + ring, bufs[j], gsem[j])
                if with_deg:
                    for j in range(ring):
                        d_wait(e0 + j)

        plsc.subcore_barrier()

        pltpu.sync_copy(h_sh.at[pl.ds(row0, RPT)], h_out.at[c, pl.ds(row0, RPT)])
        if with_deg:
            pltpu.sync_copy(deg_sh.at[pl.ds(row0, RPT)],
                            deg_out.at[c, pl.ds(row0, RPT)])

    return pl.kernel(body, out_type=tuple(out_type), mesh=mesh,
                     scratch_types=tuple(scratch),
                     compiler_params=pltpu.CompilerParams(
                         use_tc_tiling_on_sc=False))


_segsum_cache = {}


def _segsum(D, with_deg, ring):
    key = (D, with_deg, ring)
    if key not in _segsum_cache:
        _segsum_cache[key] = _make_segsum(D, with_deg, ring)
    return _segsum_cache[key]


def _mfconv_tc(hp, xin, wlf, blf, wrf, degcol, hout, relu):
    B = 2048
    din = xin.shape[1]
    kh = wlf.shape[1]
    grid = (pl.cdiv(N, B),)

    def tc_body(h0, h1, xr, wl, bl, wr, d0, d1, out):
        h = h0[0] + h1[0]
        r = jnp.dot(h, wl[...], preferred_element_type=jnp.float32)
        r += jnp.dot(xr[...], wr[...], preferred_element_type=jnp.float32)
        r += bl[...]
        deg = jnp.minimum(d0[0] + d1[0], float(K - 1))
        grp = (lax.broadcasted_iota(jnp.int32, (1, kh), 1) // hout
               ).astype(jnp.float32)
        m = r * (deg == grp).astype(jnp.float32)
        if hout % 128 == 0:
            acc = m[:, 0:hout]
            for d in range(1, K):
                acc += m[:, d * hout:(d + 1) * hout]
        else:
            row = lax.broadcasted_iota(jnp.int32, (kh, hout), 0)
            col = lax.broadcasted_iota(jnp.int32, (kh, hout), 1)
            g = (row % hout == col).astype(jnp.float32)
            acc = jnp.dot(m, g, preferred_element_type=jnp.float32)
        out[...] = jnp.maximum(acc, 0.0) if relu else acc

    return pl.pallas_call(
        tc_body,
        grid=grid,
        in_specs=[
            pl.BlockSpec((1, B, hp.shape[2]), lambda i: (0, i, 0)),
            pl.BlockSpec((1, B, hp.shape[2]), lambda i: (1, i, 0)),
            pl.BlockSpec((B, din), lambda i: (i, 0)),
            pl.BlockSpec((din, kh), lambda i: (0, 0)),
            pl.BlockSpec((1, kh), lambda i: (0, 0)),
            pl.BlockSpec((din, kh), lambda i: (0, 0)),
            pl.BlockSpec((1, B, 1), lambda i: (0, i, 0)),
            pl.BlockSpec((1, B, 1), lambda i: (1, i, 0)),
        ],
        out_specs=pl.BlockSpec((B, hout), lambda i: (i, 0)),
        out_shape=jax.ShapeDtypeStruct((N, hout), jnp.float32),
    )(hp, hp, xin, wlf, blf, wrf, degcol, degcol)


def kernel(x, edge_index, Wl1, bl1, Wr1, Wl2, bl2, Wr2):
    ei = edge_index.astype(jnp.int32)
    src = ei[0].reshape(NW, NCHUNK, C)
    dst = ei[1].reshape(NW, NCHUNK, C)

    wl1f = Wl1.transpose(1, 0, 2).reshape(128, K * 32)
    wr1f = Wr1.transpose(1, 0, 2).reshape(128, K * 32)
    bl1f = bl1.reshape(1, K * 32)
    wl2f = Wl2.transpose(1, 0, 2).reshape(32, K * 128)
    wr2f = Wr2.transpose(1, 0, 2).reshape(32, K * 128)
    bl2f = bl2.reshape(1, K * 128)

    hp1, degp = _segsum(128, True, 2)(x, src, dst)
    degcol = degp.reshape(NC, NP, 1)
    o1 = _mfconv_tc(hp1, x, wl1f, bl1f, wr1f, degcol, 32, relu=True)
    (hp2,) = _segsum(32, False, 5)(o1, src, dst)
    out = _mfconv_tc(hp2, o1, wl2f, bl2f, wr2f, degcol, 128, relu=False)
    return out


# speedup vs baseline: 1.5487x; 1.0198x over previous
import jax
import jax.numpy as jnp
from jax import lax
from jax.experimental import pallas as pl
from jax.experimental.pallas import tpu as pltpu
from jax.experimental.pallas import tpu_sc as plsc

N = 10000
NP = 10240
E = 320000
K = 11
NC, NS = 2, 16
NW = NC * NS
EPW = E // NW
C = 100
NCHUNK = EPW // C
RPT = NP // NS


def _make_segsum(D, with_deg, ring, tc_tiling=False):
    ng = NCHUNK // ring
    left = NCHUNK - ng * ring
    mesh = plsc.VectorSubcoreMesh(core_axis_name="c", subcore_axis_name="s")
    out_type = [jax.ShapeDtypeStruct((NC, NP, D), jnp.float32)]
    scratch = [
        pltpu.VMEM((NCHUNK, C), jnp.int32),
        pltpu.VMEM((NCHUNK, C), jnp.int32),
        pltpu.VMEM_SHARED((NP, D), jnp.float32),
    ]
    scratch += [pltpu.VMEM((C, D), jnp.float32) for _ in range(ring)]
    scratch += [pltpu.SemaphoreType.DMA for _ in range(2 * ring)]
    if with_deg:
        out_type.append(jax.ShapeDtypeStruct((NC, NP), jnp.float32))
        scratch += [
            pltpu.VMEM((C,), jnp.float32),
            pltpu.VMEM((RPT,), jnp.float32),
            pltpu.VMEM_SHARED((NP,), jnp.float32),
            pltpu.SemaphoreType.DMA,
        ]

    def body(x_hbm, src_hbm, dst_hbm, *rest):
        h_out = rest[0]
        rest = rest[1:]
        if with_deg:
            deg_out = rest[0]
            rest = rest[1:]
        src_all, dst_all, h_sh = rest[0], rest[1], rest[2]
        bufs = rest[3:3 + ring]
        gsem = rest[3 + ring:3 + 2 * ring]
        ssem = rest[3 + 2 * ring:3 + 3 * ring]
        if with_deg:
            ones_v, dz_v, deg_sh, dsem = rest[3 + 3 * ring:]
        c = lax.axis_index("c")
        s = lax.axis_index("s")
        wid = c * NS + s
        row0 = s * RPT

        zero16 = jnp.zeros((16,), jnp.float32)
        b0 = bufs[0]

        @pl.loop(0, C)
        def _zero_b0(r):
            for j in range(D // 16):
                b0[r, pl.ds(j * 16, 16)] = zero16

        pltpu.sync_copy(src_hbm.at[wid], src_all)
        pltpu.sync_copy(dst_hbm.at[wid], dst_all)

        @pl.loop(0, RPT // C)
        def _zero_h(k):
            pltpu.sync_copy(b0, h_sh.at[pl.ds(row0 + k * C, C)])

        if RPT % C:
            pltpu.sync_copy(b0.at[pl.ds(0, RPT % C)],
                            h_sh.at[pl.ds(row0 + (RPT // C) * C, RPT % C)])

        if with_deg:
            @pl.loop(0, RPT // 16)
            def _zero_dz(i):
                dz_v[pl.ds(i * 16, 16)] = zero16

            for j in range(C // 16):
                ones_v[pl.ds(j * 16, 16)] = jnp.ones((16,), jnp.float32)
            pltpu.sync_copy(dz_v, deg_sh.at[pl.ds(row0, RPT)])

        plsc.subcore_barrier()

        def g_issue(i, buf, sem):
            pltpu.async_copy(x_hbm.at[src_all.at[i]], buf, sem)

        def g_wait(i, buf, sem):
            pltpu.make_async_copy(x_hbm.at[src_all.at[i]], buf, sem).wait()

        def s_issue(i, buf, sem):
            pltpu.async_copy(buf, h_sh.at[dst_all.at[i]], sem, add=True)

        def s_wait(i, buf, sem):
            pltpu.make_async_copy(buf, h_sh.at[dst_all.at[i]], sem).wait()

        def d_issue(i):
            pltpu.async_copy(ones_v, deg_sh.at[dst_all.at[i]], dsem, add=True)

        def d_wait(i):
            pltpu.make_async_copy(ones_v, deg_sh.at[dst_all.at[i]], dsem).wait()

        if ring == 2:
            npair = (NCHUNK - 1) // 2
            buf0, buf1 = bufs
            gs0, gs1 = gsem
            ss0, ss1 = ssem
            g_issue(0, buf0, gs0)
            g_issue(1, buf1, gs1)

            @pl.loop(0, npair)
            def _pairs(k):
                e = k * 2
                g_wait(e, buf0, gs0)
                s_issue(e, buf0, ss0)
                if with_deg:
                    d_issue(e)
                s_wait(e, buf0, ss0)
                g_issue(e + 2, buf0, gs0)
                g_wait(e + 1, buf1, gs1)
                s_issue(e + 1, buf1, ss1)
                if with_deg:
                    d_issue(e + 1)
                s_wait(e + 1, buf1, ss1)

                @pl.when(k < npair - 1)
                def _prefetch_odd():
                    g_issue(e + 3, buf1, gs1)

                if with_deg:
                    d_wait(e)
                    d_wait(e + 1)

            last = NCHUNK - 1
            g_wait(last, buf0, gs0)
            s_issue(last, buf0, ss0)
            if with_deg:
                d_issue(last)
            s_wait(last, buf0, ss0)
            if with_deg:
                d_wait(last)
        else:
            for l in range(left):
                e = NCHUNK - 1 - l
                g_issue(e, bufs[l], gsem[l])
                g_wait(e, bufs[l], gsem[l])
                s_issue(e, bufs[l], ssem[l])
                if with_deg:
                    d_issue(e)
                    d_wait(e)
                s_wait(e, bufs[l], ssem[l])

            for j in range(ring):
                g_issue(j, bufs[j], gsem[j])

            @pl.loop(0, ng)
            def _groups(g):
                e0 = g * ring
                for j in range(ring):
                    e = e0 + j
                    g_wait(e, bufs[j], gsem[j])
                    s_issue(e, bufs[j], ssem[j])
                    if with_deg:
                        d_issue(e)
                for j in range(ring):
                    e = e0 + j
                    s_wait(e, bufs[j], ssem[j])

                    @pl.when(g < ng - 1)
                    def _next_gather():
                        g_issue(e + ring, bufs[j], gsem[j])
                if with_deg:
                    for j in range(ring):
                        d_wait(e0 + j)

        plsc.subcore_barrier()

        pltpu.sync_copy(h_sh.at[pl.ds(row0, RPT)], h_out.at[c, pl.ds(row0, RPT)])
        if with_deg:
            pltpu.sync_copy(deg_sh.at[pl.ds(row0, RPT)],
                            deg_out.at[c, pl.ds(row0, RPT)])

    return pl.kernel(body, out_type=tuple(out_type), mesh=mesh,
                     scratch_types=tuple(scratch),
                     compiler_params=pltpu.CompilerParams(
                         use_tc_tiling_on_sc=tc_tiling))


_segsum_cache = {}


def _segsum(D, with_deg, ring, tc_tiling=False):
    key = (D, with_deg, ring, tc_tiling)
    if key not in _segsum_cache:
        _segsum_cache[key] = _make_segsum(D, with_deg, ring, tc_tiling)
    return _segsum_cache[key]


def _mfconv_tc(hp, xin, wlf, blf, wrf, degcol, hout, relu):
    B = 2048
    din = xin.shape[1]
    kh = wlf.shape[1]
    grid = (pl.cdiv(N, B),)

    def tc_body(h0, h1, xr, wl, bl, wr, d0, d1, out):
        h = h0[0] + h1[0]
        r = jnp.dot(h, wl[...], preferred_element_type=jnp.float32)
        r += jnp.dot(xr[...], wr[...], preferred_element_type=jnp.float32)
        r += bl[...]
        deg = jnp.minimum(d0[0] + d1[0], float(K - 1))
        grp = (lax.broadcasted_iota(jnp.int32, (1, kh), 1) // hout
               ).astype(jnp.float32)
        m = r * (deg == grp).astype(jnp.float32)
        if hout % 128 == 0:
            acc = m[:, 0:hout]
            for d in range(1, K):
                acc += m[:, d * hout:(d + 1) * hout]
        else:
            row = lax.broadcasted_iota(jnp.int32, (kh, hout), 0)
            col = lax.broadcasted_iota(jnp.int32, (kh, hout), 1)
            g = (row % hout == col).astype(jnp.float32)
            acc = jnp.dot(m, g, preferred_element_type=jnp.float32)
        out[...] = jnp.maximum(acc, 0.0) if relu else acc

    return pl.pallas_call(
        tc_body,
        grid=grid,
        in_specs=[
            pl.BlockSpec((1, B, hp.shape[2]), lambda i: (0, i, 0)),
            pl.BlockSpec((1, B, hp.shape[2]), lambda i: (1, i, 0)),
            pl.BlockSpec((B, din), lambda i: (i, 0)),
            pl.BlockSpec((din, kh), lambda i: (0, 0)),
            pl.BlockSpec((1, kh), lambda i: (0, 0)),
            pl.BlockSpec((din, kh), lambda i: (0, 0)),
            pl.BlockSpec((1, B, 1), lambda i: (0, i, 0)),
            pl.BlockSpec((1, B, 1), lambda i: (1, i, 0)),
        ],
        out_specs=pl.BlockSpec((B, hout), lambda i: (i, 0)),
        out_shape=jax.ShapeDtypeStruct((N, hout), jnp.float32),
    )(hp, hp, xin, wlf, blf, wrf, degcol, degcol)


def kernel(x, edge_index, Wl1, bl1, Wr1, Wl2, bl2, Wr2):
    ei = edge_index.astype(jnp.int32)
    src = ei[0].reshape(NW, NCHUNK, C)
    dst = ei[1].reshape(NW, NCHUNK, C)

    wl1f = Wl1.transpose(1, 0, 2).reshape(128, K * 32)
    wr1f = Wr1.transpose(1, 0, 2).reshape(128, K * 32)
    bl1f = bl1.reshape(1, K * 32)
    wl2f = Wl2.transpose(1, 0, 2).reshape(32, K * 128)
    wr2f = Wr2.transpose(1, 0, 2).reshape(32, K * 128)
    bl2f = bl2.reshape(1, K * 128)

    hp1, degp = _segsum(128, True, 2)(x, src, dst)
    degcol = degp.reshape(NC, NP, 1)
    o1 = _mfconv_tc(hp1, x, wl1f, bl1f, wr1f, degcol, 32, relu=True)
    (hp2,) = _segsum(32, False, 5)(o1, src, dst)
    out = _mfconv_tc(hp2, o1, wl2f, bl2f, wr2f, degcol, 128, relu=False)
    return out
